# baseline (speedup 1.0000x reference)
import functools

import jax
import jax.numpy as jnp
from jax import lax
from jax.experimental import pallas as pl
from jax.experimental.pallas import tpu as pltpu
from jax.experimental.pallas import tpu_sc as plsc

N = 10000
E = 320000
D = 128
T = 12

NC = 2
NS = 16
NW = NC * NS

EB = 64
EPT_DEG = E // NW
GB = 64
BPT = 320
E_PAD = NS * BPT * EB
PAD_DST = 10008
ACC_ROWS = 10016
NBUF = 4
RPT = N // NS
CPC = T // NC

BN = 400
GRID = N // BN


def _deg_body(dst_hbm, out_hbm, acc_v, idx_v):
    c = lax.axis_index("c")
    s = lax.axis_index("s")
    wid = c * NS + s
    zero16 = jnp.zeros((16,), jnp.float32)
    ones16 = jnp.ones((16,), jnp.float32)

    def zero_step(i, carry):
        acc_v[pl.ds(i * 16, 16)] = zero16
        return carry

    lax.fori_loop(0, N // 16, zero_step, 0)

    pltpu.sync_copy(dst_hbm.at[pl.ds(wid * EPT_DEG, EPT_DEG)], idx_v)

    def step(i, carry):
        idx = idx_v[pl.ds(i * 16, 16)]
        plsc.addupdate_scatter(acc_v, [idx], ones16)
        return carry

    lax.fori_loop(0, EPT_DEG // 16, step, 0)
    pltpu.sync_copy(acc_v, out_hbm.at[wid])


def _deg_partials(dst):
    return pl.kernel(
        _deg_body,
        out_type=jax.ShapeDtypeStruct((NW, N), jnp.float32),
        mesh=plsc.VectorSubcoreMesh(
            core_axis_name="c", subcore_axis_name="s",
            num_cores=NC, num_subcores=NS),
        scratch_types=[
            pltpu.VMEM((N,), jnp.float32),
            pltpu.VMEM((EPT_DEG,), jnp.int32),
        ],
        compiler_params=pltpu.CompilerParams(
            needs_layout_passes=False, use_tc_tiling_on_sc=False),
    )(dst)


def _prescale_body(xt_ref, degp_ref, xp_ref):
    deg = jnp.sum(degp_ref[...], axis=1, keepdims=True) + 1.0
    dinv = lax.rsqrt(deg)
    xp_ref[...] = xt_ref[...] * dinv[None, :, :]


def _prescale(xt, degt):
    return pl.pallas_call(
        _prescale_body,
        grid=(GRID,),
        in_specs=[
            pl.BlockSpec((T, BN, D), lambda i: (0, i, 0)),
            pl.BlockSpec((BN, NW), lambda i: (i, 0)),
        ],
        out_specs=pl.BlockSpec((T, BN, D), lambda i: (0, i, 0)),
        out_shape=jax.ShapeDtypeStruct((T, N, D), jnp.float32),
    )(xt, degt)


def _spmm_body(xp_hbm, src2d_hbm, dst2d_hbm, zeros_hbm, out_hbm, acc_sh,
               rows4, src_grp, dst_grp,
               g0, g1, g2, g3, s0, s1, s2, s3):
    c = lax.axis_index("c")
    s = lax.axis_index("s")
    gsem = (g0, g1, g2, g3)
    ssem = (s0, s1, s2, s3)

    def gather(par, b):
        pltpu.async_copy(xp_hbm.at[src_grp.at[b]], rows4.at[par], gsem[par])

    def gather_wait(par):
        pltpu.make_async_copy(xp_hbm.at[src_grp.at[0]], rows4.at[par],
                              gsem[par]).wait()

    def scatter(par, b):
        pltpu.async_copy(rows4.at[par], acc_sh.at[dst_grp.at[b]],
                         ssem[par], add=True)

    def scatter_wait(par):
        pltpu.make_async_copy(rows4.at[par], acc_sh.at[dst_grp.at[0]],
                              ssem[par]).wait()

    def chunk_step(j, carry):
        t = c * CPC + j
        toff = (t * N).astype(jnp.int32)

        pltpu.sync_copy(zeros_hbm, acc_sh.at[pl.ds(s * RPT, RPT)])
        plsc.subcore_barrier()

        def group_step(g, carry2):
            grow = s * BPT + g * GB
            pltpu.sync_copy(src2d_hbm.at[pl.ds(grow, GB)], src_grp)
            pltpu.sync_copy(dst2d_hbm.at[pl.ds(grow, GB)], dst_grp)

            def shift(r, carry3):
                for k in range(EB // 16):
                    src_grp[r, pl.ds(k * 16, 16)] = (
                        src_grp[r, pl.ds(k * 16, 16)] + toff)
                return carry3

            lax.fori_loop(0, GB, shift, 0)

            for par in range(NBUF):
                gather(par, par)

            def pipe_step(i, carry3):
                b0 = i * NBUF
                for par in range(NBUF):
                    gather_wait(par)
                    scatter(par, b0 + par)
                for par in range(NBUF):
                    scatter_wait(par)

                    @pl.when(b0 + par + NBUF < GB)
                    def _():
                        gather(par, b0 + par + NBUF)

                return carry3

            lax.fori_loop(0, GB // NBUF, pipe_step, 0)
            return carry2

        lax.fori_loop(0, BPT // GB, group_step, 0)
        plsc.subcore_barrier()
        pltpu.sync_copy(
            acc_sh.at[pl.ds(s * RPT, RPT)],
            out_hbm.at[pl.ds(t * N + s * RPT, RPT)])
        return carry

    lax.fori_loop(0, CPC, chunk_step, 0)


def _spmm(xp_flat, src2d, dst2d, zeros):
    return pl.kernel(
        _spmm_body,
        out_type=jax.ShapeDtypeStruct((T * N, D), jnp.float32),
        mesh=plsc.VectorSubcoreMesh(
            core_axis_name="c", subcore_axis_name="s",
            num_cores=NC, num_subcores=NS),
        scratch_types=[
            pltpu.VMEM_SHARED((ACC_ROWS, D), jnp.float32),
            pltpu.VMEM((NBUF, EB, D), jnp.float32),
            pltpu.VMEM((GB, EB), jnp.int32),
            pltpu.VMEM((GB, EB), jnp.int32),
            pltpu.SemaphoreType.DMA,
            pltpu.SemaphoreType.DMA,
            pltpu.SemaphoreType.DMA,
            pltpu.SemaphoreType.DMA,
            pltpu.SemaphoreType.DMA,
            pltpu.SemaphoreType.DMA,
            pltpu.SemaphoreType.DMA,
            pltpu.SemaphoreType.DMA,
        ],
        compiler_params=pltpu.CompilerParams(
            needs_layout_passes=False, use_tc_tiling_on_sc=False),
    )(xp_flat, src2d, dst2d, zeros)


def _dense_body(s_ref, xp_ref, degp_ref, wz_ref, lz_ref, wh_ref, lh_ref,
                bz_ref, lzb_ref, bh_ref, lhb_ref, att_ref, wo_ref, bo_ref,
                out_ref):
    deg = jnp.sum(degp_ref[...], axis=1, keepdims=True) + 1.0
    dinv = lax.rsqrt(deg)

    att = att_ref[...]
    m = jnp.max(att, axis=1, keepdims=True)
    ea = jnp.exp(att - m)
    p = ea / jnp.sum(ea, axis=1, keepdims=True)

    wz = wz_ref[...]
    lz = lz_ref[...]
    wh = wh_ref[...]
    lh = lh_ref[...]
    bz = bz_ref[...]
    lzb = lzb_ref[...]
    bh = bh_ref[...]
    lhb = lhb_ref[...]

    acc = jnp.zeros((BN, D), jnp.float32)
    for t in range(T):
        ax = dinv * (s_ref[t] + xp_ref[t])
        gz = jnp.dot(ax, wz, preferred_element_type=jnp.float32) + bz
        gz = jnp.dot(gz, lz, preferred_element_type=jnp.float32) + lzb
        gh = jnp.dot(ax, wh, preferred_element_type=jnp.float32) + bh
        gh = jnp.dot(gh, lh, preferred_element_type=jnp.float32) + lhb
        h = (1.0 - jax.nn.sigmoid(gz)) * jnp.tanh(gh)
        acc = acc + p[0, t] * h

    out_ref[...] = (jnp.dot(jax.nn.relu(acc), wo_ref[...],
                            preferred_element_type=jnp.float32) + bo_ref[...])


def _dense(s3, xp, degt, wz, lz, wh, lh, bz, lzb, bh, lhb, att2, wo, bo):
    def full(shape):
        nd = len(shape)
        return pl.BlockSpec(shape, lambda i, _nd=nd: (0,) * _nd)
    return pl.pallas_call(
        _dense_body,
        grid=(GRID,),
        in_specs=[
            pl.BlockSpec((T, BN, D), lambda i: (0, i, 0)),
            pl.BlockSpec((T, BN, D), lambda i: (0, i, 0)),
            pl.BlockSpec((BN, NW), lambda i: (i, 0)),
            full((D, D)), full((D, D)), full((D, D)), full((D, D)),
            full((1, D)), full((1, D)), full((1, D)), full((1, D)),
            full((1, T)), full((D, T)), full((1, T)),
        ],
        out_specs=pl.BlockSpec((BN, T), lambda i: (i, 0)),
        out_shape=jax.ShapeDtypeStruct((N, T), jnp.float32),
    )(s3, xp, degt, wz, lz, wh, lh, bz, lzb, bh, lhb, att2, wo, bo)


def kernel(x_1, edge_index_1, x_2, edge_index_2, W_z, b_z, W_r, b_r, W_h, b_h,
           lin_z_W, lin_z_b, lin_r_W, lin_r_b, lin_h_W, lin_h_b, att, W_out,
           b_out):
    src = edge_index_1[0]
    dst = edge_index_1[1]

    xt = jnp.transpose(x_1, (2, 0, 1))
    npad = E_PAD - E
    src2d = jnp.concatenate(
        [src, jnp.zeros((npad,), jnp.int32)]).reshape(E_PAD // EB, EB)
    dst2d = jnp.concatenate(
        [dst, jnp.full((npad,), PAD_DST, jnp.int32)]).reshape(E_PAD // EB, EB)

    degp = _deg_partials(dst)
    degt = jnp.transpose(degp)

    xp = _prescale(xt, degt)
    zeros = jnp.zeros((RPT, D), jnp.float32)
    s_flat = _spmm(xp.reshape(T * N, D), src2d, dst2d, zeros)
    s3 = s_flat.reshape(T, N, D)

    return _dense(
        s3, xp, degt,
        W_z, lin_z_W[:D], W_h, lin_h_W[:D],
        b_z.reshape(1, D), lin_z_b.reshape(1, D),
        b_h.reshape(1, D), lin_h_b.reshape(1, D),
        att.reshape(1, T), W_out, b_out.reshape(1, T))


# speedup vs baseline: 1.0027x; 1.0027x over previous
import functools

import jax
import jax.numpy as jnp
from jax import lax
from jax.experimental import pallas as pl
from jax.experimental.pallas import tpu as pltpu
from jax.experimental.pallas import tpu_sc as plsc

N = 10000
E = 320000
D = 128
T = 12

NC = 2
NS = 16
NW = NC * NS

EB = 64
EPT_DEG = E // NW
GB = 32
BPT = 320
E_PAD = NS * BPT * EB
PAD_DST = 10008
ACC_ROWS = 10016
NBUF = 4
RPT = N // NS
CPC = T // NC

BN = 400
GRID = N // BN


def _deg_body(dst_hbm, out_hbm, acc_v, idx_v):
    c = lax.axis_index("c")
    s = lax.axis_index("s")
    wid = c * NS + s
    zero16 = jnp.zeros((16,), jnp.float32)
    ones16 = jnp.ones((16,), jnp.float32)

    def zero_step(i, carry):
        acc_v[pl.ds(i * 16, 16)] = zero16
        return carry

    lax.fori_loop(0, N // 16, zero_step, 0)

    pltpu.sync_copy(dst_hbm.at[pl.ds(wid * EPT_DEG, EPT_DEG)], idx_v)

    def step(i, carry):
        idx = idx_v[pl.ds(i * 16, 16)]
        plsc.addupdate_scatter(acc_v, [idx], ones16)
        return carry

    lax.fori_loop(0, EPT_DEG // 16, step, 0)
    pltpu.sync_copy(acc_v, out_hbm.at[wid])


def _deg_partials(dst):
    return pl.kernel(
        _deg_body,
        out_type=jax.ShapeDtypeStruct((NW, N), jnp.float32),
        mesh=plsc.VectorSubcoreMesh(
            core_axis_name="c", subcore_axis_name="s",
            num_cores=NC, num_subcores=NS),
        scratch_types=[
            pltpu.VMEM((N,), jnp.float32),
            pltpu.VMEM((EPT_DEG,), jnp.int32),
        ],
        compiler_params=pltpu.CompilerParams(
            needs_layout_passes=False, use_tc_tiling_on_sc=False),
    )(dst)


def _prescale_body(xt_ref, degp_ref, xp_ref):
    deg = jnp.sum(degp_ref[...], axis=1, keepdims=True) + 1.0
    dinv = lax.rsqrt(deg)
    xp_ref[...] = xt_ref[...] * dinv[None, :, :]


def _prescale(xt, degt):
    return pl.pallas_call(
        _prescale_body,
        grid=(GRID,),
        in_specs=[
            pl.BlockSpec((T, BN, D), lambda i: (0, i, 0)),
            pl.BlockSpec((BN, NW), lambda i: (i, 0)),
        ],
        out_specs=pl.BlockSpec((T, BN, D), lambda i: (0, i, 0)),
        out_shape=jax.ShapeDtypeStruct((T, N, D), jnp.float32),
    )(xt, degt)


def _spmm_body(xp_hbm, src2d_hbm, dst2d_hbm, zeros_hbm, out_hbm, acc_sh,
               rows4, src_grp, dst_grp,
               g0, g1, g2, g3, s0, s1, s2, s3):
    c = lax.axis_index("c")
    s = lax.axis_index("s")
    gsem = (g0, g1, g2, g3)
    ssem = (s0, s1, s2, s3)

    def gather(par, b):
        slot = (b // GB) % 2
        grow = b % GB
        pltpu.async_copy(xp_hbm.at[src_grp.at[slot, grow]], rows4.at[par],
                         gsem[par])

    def gather_wait(par):
        pltpu.make_async_copy(xp_hbm.at[src_grp.at[0, 0]], rows4.at[par],
                              gsem[par]).wait()

    def scatter(par, b):
        slot = (b // GB) % 2
        grow = b % GB
        pltpu.async_copy(rows4.at[par], acc_sh.at[dst_grp.at[slot, grow]],
                         ssem[par], add=True)

    def scatter_wait(par):
        pltpu.make_async_copy(rows4.at[par], acc_sh.at[dst_grp.at[0, 0]],
                              ssem[par]).wait()

    def chunk_step(j, carry):
        t = c * CPC + j
        toff = (t * N).astype(jnp.int32)

        def stage(gi):
            slot = gi % 2
            grow = s * BPT + gi * GB
            pltpu.sync_copy(src2d_hbm.at[pl.ds(grow, GB)], src_grp.at[slot])
            pltpu.sync_copy(dst2d_hbm.at[pl.ds(grow, GB)], dst_grp.at[slot])

            def shift(r, carry2):
                for k in range(EB // 16):
                    src_grp[slot, r, pl.ds(k * 16, 16)] = (
                        src_grp[slot, r, pl.ds(k * 16, 16)] + toff)
                return carry2

            lax.fori_loop(0, GB, shift, 0)

        pltpu.sync_copy(zeros_hbm, acc_sh.at[pl.ds(s * RPT, RPT)])
        plsc.subcore_barrier()

        stage(0)

        def pipe_step(i, carry2):
            for par in range(NBUF):
                b = i * NBUF + par
                d = b - 2
                e = b - 4
                if par == 0:
                    @pl.when((lax.rem(i, GB // NBUF) == 0) & (i > 0)
                             & (b < BPT))
                    def _():
                        stage(b // GB)

                gpar = (par + 2) % NBUF

                @pl.when((d >= 0) & (d < BPT))
                def _():
                    gather_wait(gpar)
                    scatter(gpar, d)

                @pl.when((e >= 0) & (e < BPT))
                def _():
                    scatter_wait(par)

                @pl.when(b < BPT)
                def _():
                    gather(par, b)

            return carry2

        lax.fori_loop(0, BPT // NBUF + 1, pipe_step, 0)
        plsc.subcore_barrier()
        pltpu.sync_copy(
            acc_sh.at[pl.ds(s * RPT, RPT)],
            out_hbm.at[pl.ds(t * N + s * RPT, RPT)])
        return carry

    lax.fori_loop(0, CPC, chunk_step, 0)


def _spmm(xp_flat, src2d, dst2d, zeros):
    return pl.kernel(
        _spmm_body,
        out_type=jax.ShapeDtypeStruct((T * N, D), jnp.float32),
        mesh=plsc.VectorSubcoreMesh(
            core_axis_name="c", subcore_axis_name="s",
            num_cores=NC, num_subcores=NS),
        scratch_types=[
            pltpu.VMEM_SHARED((ACC_ROWS, D), jnp.float32),
            pltpu.VMEM((NBUF, EB, D), jnp.float32),
            pltpu.VMEM((2, GB, EB), jnp.int32),
            pltpu.VMEM((2, GB, EB), jnp.int32),
            pltpu.SemaphoreType.DMA,
            pltpu.SemaphoreType.DMA,
            pltpu.SemaphoreType.DMA,
            pltpu.SemaphoreType.DMA,
            pltpu.SemaphoreType.DMA,
            pltpu.SemaphoreType.DMA,
            pltpu.SemaphoreType.DMA,
            pltpu.SemaphoreType.DMA,
        ],
        compiler_params=pltpu.CompilerParams(
            needs_layout_passes=False, use_tc_tiling_on_sc=False),
    )(xp_flat, src2d, dst2d, zeros)


def _dense_body(s_ref, xp_ref, degp_ref, wz_ref, lz_ref, wh_ref, lh_ref,
                bz_ref, lzb_ref, bh_ref, lhb_ref, att_ref, wo_ref, bo_ref,
                out_ref):
    deg = jnp.sum(degp_ref[...], axis=1, keepdims=True) + 1.0
    dinv = lax.rsqrt(deg)

    att = att_ref[...]
    m = jnp.max(att, axis=1, keepdims=True)
    ea = jnp.exp(att - m)
    p = ea / jnp.sum(ea, axis=1, keepdims=True)

    wz = wz_ref[...]
    lz = lz_ref[...]
    wh = wh_ref[...]
    lh = lh_ref[...]
    bz = bz_ref[...]
    lzb = lzb_ref[...]
    bh = bh_ref[...]
    lhb = lhb_ref[...]

    acc = jnp.zeros((BN, D), jnp.float32)
    for t in range(T):
        ax = dinv * (s_ref[t] + xp_ref[t])
        gz = jnp.dot(ax, wz, preferred_element_type=jnp.float32) + bz
        gz = jnp.dot(gz, lz, preferred_element_type=jnp.float32) + lzb
        gh = jnp.dot(ax, wh, preferred_element_type=jnp.float32) + bh
        gh = jnp.dot(gh, lh, preferred_element_type=jnp.float32) + lhb
        h = (1.0 - jax.nn.sigmoid(gz)) * jnp.tanh(gh)
        acc = acc + p[0, t] * h

    out_ref[...] = (jnp.dot(jax.nn.relu(acc), wo_ref[...],
                            preferred_element_type=jnp.float32) + bo_ref[...])


def _dense(s3, xp, degt, wz, lz, wh, lh, bz, lzb, bh, lhb, att2, wo, bo):
    def full(shape):
        nd = len(shape)
        return pl.BlockSpec(shape, lambda i, _nd=nd: (0,) * _nd)
    return pl.pallas_call(
        _dense_body,
        grid=(GRID,),
        in_specs=[
            pl.BlockSpec((T, BN, D), lambda i: (0, i, 0)),
            pl.BlockSpec((T, BN, D), lambda i: (0, i, 0)),
            pl.BlockSpec((BN, NW), lambda i: (i, 0)),
            full((D, D)), full((D, D)), full((D, D)), full((D, D)),
            full((1, D)), full((1, D)), full((1, D)), full((1, D)),
            full((1, T)), full((D, T)), full((1, T)),
        ],
        out_specs=pl.BlockSpec((BN, T), lambda i: (i, 0)),
        out_shape=jax.ShapeDtypeStruct((N, T), jnp.float32),
    )(s3, xp, degt, wz, lz, wh, lh, bz, lzb, bh, lhb, att2, wo, bo)


def kernel(x_1, edge_index_1, x_2, edge_index_2, W_z, b_z, W_r, b_r, W_h, b_h,
           lin_z_W, lin_z_b, lin_r_W, lin_r_b, lin_h_W, lin_h_b, att, W_out,
           b_out):
    src = edge_index_1[0]
    dst = edge_index_1[1]

    xt = jnp.transpose(x_1, (2, 0, 1))
    npad = E_PAD - E
    src2d = jnp.concatenate(
        [src, jnp.zeros((npad,), jnp.int32)]).reshape(E_PAD // EB, EB)
    dst2d = jnp.concatenate(
        [dst, jnp.full((npad,), PAD_DST, jnp.int32)]).reshape(E_PAD // EB, EB)

    degp = _deg_partials(dst)
    degt = jnp.transpose(degp)

    xp = _prescale(xt, degt)
    zeros = jnp.zeros((RPT, D), jnp.float32)
    s_flat = _spmm(xp.reshape(T * N, D), src2d, dst2d, zeros)
    s3 = s_flat.reshape(T, N, D)

    return _dense(
        s3, xp, degt,
        W_z, lin_z_W[:D], W_h, lin_h_W[:D],
        b_z.reshape(1, D), lin_z_b.reshape(1, D),
        b_h.reshape(1, D), lin_h_b.reshape(1, D),
        att.reshape(1, T), W_out, b_out.reshape(1, T))


# speedup vs baseline: 2.3998x; 2.3934x over previous
import functools

import jax
import jax.numpy as jnp
from jax import lax
from jax.experimental import pallas as pl
from jax.experimental.pallas import tpu as pltpu
from jax.experimental.pallas import tpu_sc as plsc

N = 10000
E = 320000
D = 128
T = 12

NC = 2
NS = 16
NW = NC * NS

EB = 80
EPT_DEG = E // NW
GB = 50
BPT = E // (NS * EB)
ACC_ROWS = N
RPT = N // NS
CPC = T // NC

BN = 400
GRID = N // BN


def _deg_body(dst_hbm, out_hbm, acc_v, idx_v):
    c = lax.axis_index("c")
    s = lax.axis_index("s")
    wid = c * NS + s
    zero16 = jnp.zeros((16,), jnp.float32)
    ones16 = jnp.ones((16,), jnp.float32)

    def zero_step(i, carry):
        acc_v[pl.ds(i * 16, 16)] = zero16
        return carry

    lax.fori_loop(0, N // 16, zero_step, 0)

    pltpu.sync_copy(dst_hbm.at[pl.ds(wid * EPT_DEG, EPT_DEG)], idx_v)

    def step(i, carry):
        idx = idx_v[pl.ds(i * 16, 16)]
        plsc.addupdate_scatter(acc_v, [idx], ones16)
        return carry

    lax.fori_loop(0, EPT_DEG // 16, step, 0)
    pltpu.sync_copy(acc_v, out_hbm.at[wid])


def _deg_partials(dst):
    return pl.kernel(
        _deg_body,
        out_type=jax.ShapeDtypeStruct((NW, N), jnp.float32),
        mesh=plsc.VectorSubcoreMesh(
            core_axis_name="c", subcore_axis_name="s",
            num_cores=NC, num_subcores=NS),
        scratch_types=[
            pltpu.VMEM((N,), jnp.float32),
            pltpu.VMEM((EPT_DEG,), jnp.int32),
        ],
        compiler_params=pltpu.CompilerParams(
            needs_layout_passes=False, use_tc_tiling_on_sc=False),
    )(dst)


def _prescale_body(xt_ref, degp_ref, xp_ref):
    deg = jnp.sum(degp_ref[...], axis=1, keepdims=True) + 1.0
    dinv = lax.rsqrt(deg)
    xp_ref[...] = xt_ref[...] * dinv[None, :, :]


def _prescale(xt, degt):
    return pl.pallas_call(
        _prescale_body,
        grid=(GRID,),
        in_specs=[
            pl.BlockSpec((T, BN, D), lambda i: (0, i, 0)),
            pl.BlockSpec((BN, NW), lambda i: (i, 0)),
        ],
        out_specs=pl.BlockSpec((T, BN, D), lambda i: (0, i, 0)),
        out_shape=jax.ShapeDtypeStruct((T, N, D), jnp.float32),
    )(xt, degt)


def _spmm_body(xp_hbm, src2d_hbm, dst2d_hbm, zeros_hbm, out_hbm, acc_sh,
               rows2, src_grp, dst_grp, g0, g1):
    c = lax.axis_index("c")
    s = lax.axis_index("s")
    gsem = (g0, g1)

    def chunk_step(j, carry):
        t = c * CPC + j
        toff = (t * N).astype(jnp.int32)

        def stage_group(gi):
            slot = gi % 2
            grow = s * BPT + gi * GB
            pltpu.sync_copy(src2d_hbm.at[pl.ds(grow, GB)], src_grp.at[slot])
            pltpu.sync_copy(dst2d_hbm.at[pl.ds(grow, GB)], dst_grp.at[slot])

            def shift(r, carry2):
                for k in range(EB // 16):
                    src_grp[slot, r, pl.ds(k * 16, 16)] = (
                        src_grp[slot, r, pl.ds(k * 16, 16)] + toff)
                return carry2

            lax.fori_loop(0, GB, shift, 0)

        def gather(par, b):
            slot = (b // GB) % 2
            grow = b % GB
            pltpu.async_copy(xp_hbm.at[src_grp.at[slot, grow]],
                             rows2.at[par], gsem[par])

        def consume(par, b):
            slot = (b // GB) % 2
            grow = b % GB
            pltpu.make_async_copy(xp_hbm.at[src_grp.at[0, 0]],
                                  rows2.at[par], gsem[par]).wait()
            pltpu.sync_copy(rows2.at[par], acc_sh.at[dst_grp.at[slot, grow]],
                            add=True)

        pltpu.sync_copy(zeros_hbm, acc_sh.at[pl.ds(s * RPT, RPT)])
        plsc.subcore_barrier()

        stage_group(0)
        gather(0, 0)

        def pair_step(i, carry2):
            b = 2 * i
            gather(1, b + 1)
            consume(0, b)

            @pl.when((lax.rem(b + 2, GB) == 0) & (b + 2 < BPT))
            def _():
                stage_group((b + 2) // GB)

            @pl.when(b + 2 < BPT)
            def _():
                gather(0, b + 2)

            consume(1, b + 1)
            return carry2

        lax.fori_loop(0, BPT // 2, pair_step, 0)
        plsc.subcore_barrier()
        pltpu.sync_copy(
            acc_sh.at[pl.ds(s * RPT, RPT)],
            out_hbm.at[pl.ds(t * N + s * RPT, RPT)])
        return carry

    lax.fori_loop(0, CPC, chunk_step, 0)


def _spmm(xp_flat, src2d, dst2d, zeros):
    return pl.kernel(
        _spmm_body,
        out_type=jax.ShapeDtypeStruct((T * N, D), jnp.float32),
        mesh=plsc.VectorSubcoreMesh(
            core_axis_name="c", subcore_axis_name="s",
            num_cores=NC, num_subcores=NS),
        scratch_types=[
            pltpu.VMEM_SHARED((ACC_ROWS, D), jnp.float32),
            pltpu.VMEM((2, EB, D), jnp.float32),
            pltpu.VMEM((2, GB, EB), jnp.int32),
            pltpu.VMEM((2, GB, EB), jnp.int32),
            pltpu.SemaphoreType.DMA,
            pltpu.SemaphoreType.DMA,
        ],
        compiler_params=pltpu.CompilerParams(
            needs_layout_passes=False, use_tc_tiling_on_sc=False),
    )(xp_flat, src2d, dst2d, zeros)


def _dense_body(s_ref, xp_ref, degp_ref, wz_ref, lz_ref, wh_ref, lh_ref,
                bz_ref, lzb_ref, bh_ref, lhb_ref, att_ref, wo_ref, bo_ref,
                out_ref):
    deg = jnp.sum(degp_ref[...], axis=1, keepdims=True) + 1.0
    dinv = lax.rsqrt(deg)

    att = att_ref[...]
    m = jnp.max(att, axis=1, keepdims=True)
    ea = jnp.exp(att - m)
    p = ea / jnp.sum(ea, axis=1, keepdims=True)

    wz = wz_ref[...]
    lz = lz_ref[...]
    wh = wh_ref[...]
    lh = lh_ref[...]
    bz = bz_ref[...]
    lzb = lzb_ref[...]
    bh = bh_ref[...]
    lhb = lhb_ref[...]

    acc = jnp.zeros((BN, D), jnp.float32)
    for t in range(T):
        ax = dinv * (s_ref[t] + xp_ref[t])
        gz = jnp.dot(ax, wz, preferred_element_type=jnp.float32) + bz
        gz = jnp.dot(gz, lz, preferred_element_type=jnp.float32) + lzb
        gh = jnp.dot(ax, wh, preferred_element_type=jnp.float32) + bh
        gh = jnp.dot(gh, lh, preferred_element_type=jnp.float32) + lhb
        h = (1.0 - jax.nn.sigmoid(gz)) * jnp.tanh(gh)
        acc = acc + p[0, t] * h

    out_ref[...] = (jnp.dot(jax.nn.relu(acc), wo_ref[...],
                            preferred_element_type=jnp.float32) + bo_ref[...])


def _dense(s3, xp, degt, wz, lz, wh, lh, bz, lzb, bh, lhb, att2, wo, bo):
    def full(shape):
        nd = len(shape)
        return pl.BlockSpec(shape, lambda i, _nd=nd: (0,) * _nd)
    return pl.pallas_call(
        _dense_body,
        grid=(GRID,),
        in_specs=[
            pl.BlockSpec((T, BN, D), lambda i: (0, i, 0)),
            pl.BlockSpec((T, BN, D), lambda i: (0, i, 0)),
            pl.BlockSpec((BN, NW), lambda i: (i, 0)),
            full((D, D)), full((D, D)), full((D, D)), full((D, D)),
            full((1, D)), full((1, D)), full((1, D)), full((1, D)),
            full((1, T)), full((D, T)), full((1, T)),
        ],
        out_specs=pl.BlockSpec((BN, T), lambda i: (i, 0)),
        out_shape=jax.ShapeDtypeStruct((N, T), jnp.float32),
    )(s3, xp, degt, wz, lz, wh, lh, bz, lzb, bh, lhb, att2, wo, bo)


def kernel(x_1, edge_index_1, x_2, edge_index_2, W_z, b_z, W_r, b_r, W_h, b_h,
           lin_z_W, lin_z_b, lin_r_W, lin_r_b, lin_h_W, lin_h_b, att, W_out,
           b_out):
    src = edge_index_1[0]
    dst = edge_index_1[1]

    xt = jnp.transpose(x_1, (2, 0, 1))
    src2d = src.reshape(E // EB, EB)
    dst2d = dst.reshape(E // EB, EB)

    degp = _deg_partials(dst)
    degt = jnp.transpose(degp)

    xp = _prescale(xt, degt)
    zeros = jnp.zeros((RPT, D), jnp.float32)
    s_flat = _spmm(xp.reshape(T * N, D), src2d, dst2d, zeros)
    s3 = s_flat.reshape(T, N, D)

    return _dense(
        s3, xp, degt,
        W_z, lin_z_W[:D], W_h, lin_h_W[:D],
        b_z.reshape(1, D), lin_z_b.reshape(1, D),
        b_h.reshape(1, D), lin_h_b.reshape(1, D),
        att.reshape(1, T), W_out, b_out.reshape(1, T))
